# baseline (speedup 1.0000x reference)
import functools

import jax
import jax.numpy as jnp
from jax import lax
from jax.experimental import pallas as pl
from jax.experimental.pallas import tpu as pltpu
from jax.experimental.pallas import tpu_sc as plsc

D = 768
L = 16
NC, NS = 2, 16
NW = NC * NS
CHUNK = 16
NSLOTS = 4
AHEAD = 2


def _embed_body(word_hbm, pos_hbm, wi_hbm, pi_hbm, out_hbm,
                idxw_v, idxp_v,
                bufw0, bufw1, bufw2, bufw3,
                bufp0, bufp1, bufp2, bufp3,
                semw0, semw1, semw2, semw3,
                semp0, semp1, semp2, semp3,
                semst0, semst1, semst2, semst3):
    wid = lax.axis_index("s") * NC + lax.axis_index("c")
    rows_per_w = out_hbm.shape[0] // NW
    n_chunks = rows_per_w // CHUNK
    base = wid * rows_per_w

    bufw = (bufw0, bufw1, bufw2, bufw3)
    bufp = (bufp0, bufp1, bufp2, bufp3)
    semw = (semw0, semw1, semw2, semw3)
    semp = (semp0, semp1, semp2, semp3)
    semst = (semst0, semst1, semst2, semst3)

    pltpu.sync_copy(wi_hbm.at[pl.ds(base, rows_per_w)], idxw_v)
    pltpu.sync_copy(pi_hbm.at[pl.ds(base, rows_per_w)], idxp_v)

    def widx(k):
        return idxw_v.at[pl.ds(k * CHUNK, CHUNK)]

    def pidx(k):
        return idxp_v.at[pl.ds(k * CHUNK, CHUNK)]

    def fire_gathers(k, slot):
        pltpu.async_copy(word_hbm.at[widx(k)], bufw[slot], semw[slot])
        pltpu.async_copy(pos_hbm.at[pidx(k)], bufp[slot], semp[slot])

    def out_at(k):
        return out_hbm.at[pl.ds(base + k * CHUNK, CHUNK)]

    for k in range(AHEAD):
        fire_gathers(k, k)

    for g in range(n_chunks):
        s = g % NSLOTS
        nxt = g + AHEAD
        if nxt < n_chunks:
            ns = nxt % NSLOTS
            prev = nxt - NSLOTS
            if prev >= 0:
                pltpu.make_async_copy(bufw[ns], out_at(prev), semst[ns]).wait()
            fire_gathers(nxt, ns)
        pltpu.make_async_copy(word_hbm.at[widx(g)], bufw[s], semw[s]).wait()
        pltpu.make_async_copy(pos_hbm.at[pidx(g)], bufp[s], semp[s]).wait()

        def add_row(r, _, s=s):
            for c in range(D // L):
                sl = pl.ds(c * L, L)
                plsc.addupdate(bufw[s].at[r, sl], bufp[s][r, sl])
            return _

        lax.fori_loop(0, CHUNK, add_row, 0)
        pltpu.async_copy(bufw[s], out_at(g), semst[s])

    for g in range(n_chunks - NSLOTS, n_chunks):
        if g >= 0:
            s = g % NSLOTS
            pltpu.make_async_copy(bufw[s], out_at(g), semst[s]).wait()


@functools.partial(jax.jit, static_argnums=())
def _embed(word_embeddings, position_embeddings, wi, pi):
    n = wi.shape[0]
    rows_per_w = n // NW
    mesh = plsc.VectorSubcoreMesh(core_axis_name="c", subcore_axis_name="s",
                                  num_cores=NC, num_subcores=NS)
    return pl.kernel(
        _embed_body,
        out_type=jax.ShapeDtypeStruct((n, D), jnp.float32),
        mesh=mesh,
        scratch_types=(
            [pltpu.VMEM((rows_per_w,), jnp.int32)] * 2
            + [pltpu.VMEM((CHUNK, D), jnp.float32)] * (2 * NSLOTS)
            + [pltpu.SemaphoreType.DMA] * (3 * NSLOTS)
        ),
    )(word_embeddings, position_embeddings, wi, pi)


def kernel(input_ids, position_ids, word_embeddings, position_embeddings):
    B, S = input_ids.shape
    wi = input_ids.reshape(-1).astype(jnp.int32)
    pi = position_ids.reshape(-1).astype(jnp.int32)
    out = _embed(word_embeddings, position_embeddings, wi, pi)
    return out.reshape(B, S, D)
